# Initial kernel scaffold; baseline (speedup 1.0000x reference)
import jax
import jax.numpy as jnp
from jax.experimental import pallas as pl


def kernel(x, edge_index, batch, params):
    raise NotImplementedError("write your pallas kernel here")


# speedup vs baseline: 2.6442x; 2.6442x over previous
import functools

import numpy as np
import jax
import jax.numpy as jnp
from jax import lax
from jax.experimental import pallas as pl
from jax.experimental.pallas import tpu as pltpu
from jax.experimental.pallas import tpu_sc as plsc

_N, _E, _D, _H, _G, _K = 10000, 320000, 128, 32, 512, 7
_KM_ITERS = 10
_THRESH = 1.0
_BN_DEN = float(np.sqrt(np.float32(1.0 + 1e-5), dtype=np.float32))

_NC, _NS = 2, 16
_NW = _NC * _NS
_CH = 80

_F32 = jnp.float32


@functools.cache
def _make_seg_sum(acc_rows, width, epad):
  rpw = acc_rows // _NW
  assert rpw % 8 == 0 and epad % 8 == 0

  mesh = plsc.VectorSubcoreMesh(core_axis_name="c", subcore_axis_name="s")

  @functools.partial(
      pl.kernel,
      out_type=jax.ShapeDtypeStruct((acc_rows, width), _F32),
      mesh=mesh,
      scratch_types=[
          pltpu.VMEM((_CH,), jnp.int32),
          pltpu.VMEM((_CH,), jnp.int32),
          pltpu.VMEM((_CH, width), _F32),
          pltpu.VMEM((80,), jnp.int32),
          pltpu.VMEM((rpw + 8, width), _F32),
          pltpu.SemaphoreType.DMA,
      ],
      compiler_params=pltpu.CompilerParams(use_tc_tiling_on_sc=False),
  )
  def seg_sum(vals_hbm, ss_hbm, sd_hbm, meta_hbm, out_hbm,
              ssv, sdv, rows, meta_v, acc, sem):
    c = lax.axis_index("c")
    s = lax.axis_index("s")
    w = c * _NS + s
    lo = w * rpw

    pltpu.sync_copy(meta_hbm, meta_v)
    astart = meta_v[pl.ds(w, 16)][0]
    nch = meta_v[pl.ds(32 + w, 16)][0]

    def zero_row(i, carry):
      for j in range(width // 16):
        acc[i, pl.ds(16 * j, 16)] = jnp.zeros((16,), _F32)
      return carry

    lax.fori_loop(0, rpw + 8, zero_row, 0)

    def chunk(i, carry):
      off = pl.multiple_of(astart + i * _CH, 8)
      pltpu.sync_copy(ss_hbm.at[pl.ds(off, _CH)], ssv)
      pltpu.sync_copy(sd_hbm.at[pl.ds(off, _CH)], sdv)
      pltpu.async_copy(vals_hbm.at[ssv], rows, sem).wait()

      def group(g, carry2):
        dvec = sdv[pl.ds(g * 16, 16)] - lo
        okv = jnp.logical_and(dvec >= 0, dvec < rpw)
        dlv = jnp.where(okv, dvec, rpw)
        for j in range(16):
          dl = dlv[j]
          e = g * 16 + j
          for k in range(width // 16):
            acc[dl, pl.ds(16 * k, 16)] = (
                acc[dl, pl.ds(16 * k, 16)] + rows[e, pl.ds(16 * k, 16)])
        return carry2

      lax.fori_loop(0, _CH // 16, group, 0)
      return carry

    lax.fori_loop(0, nch, chunk, 0)
    pltpu.sync_copy(acc.at[pl.ds(0, rpw)], out_hbm.at[pl.ds(lo, rpw)])

  return seg_sum


_ACCN = 10240


def _seg_sum_sorted(vals, ss_sorted, sd_sorted, acc_rows, width):
  n_idx = sd_sorted.shape[0]
  ss_p = jnp.concatenate(
      [ss_sorted, jnp.zeros((_CH,), jnp.int32)])
  sd_p = jnp.concatenate(
      [sd_sorted, jnp.full((_CH,), acc_rows, jnp.int32)])
  rpw = acc_rows // _NW
  starts = jnp.searchsorted(
      sd_sorted, jnp.arange(33, dtype=jnp.int32) * rpw).astype(jnp.int32)
  astart = (starts[:32] // 8) * 8
  aend = jnp.minimum(((starts[1:] + 7) // 8) * 8, n_idx)
  nch = jnp.maximum((aend - astart + _CH - 1) // _CH, 0).astype(jnp.int32)
  meta = jnp.concatenate(
      [astart, nch, jnp.zeros((16,), jnp.int32)]).astype(jnp.int32)
  return _make_seg_sum(acc_rows, width, n_idx + _CH)(vals, ss_p, sd_p, meta)


_BLK = 1000
_NBLK = _N // _BLK


def _layer_body(h_ref, agg_ref, w1_ref, b1_ref, g_ref, be_ref, w2_ref,
                b2_ref, o_ref):
  hs = h_ref[...] + agg_ref[...]
  t = jnp.dot(hs, w1_ref[...], preferred_element_type=_F32) + b1_ref[...]
  t = g_ref[...] * t / _BN_DEN + be_ref[...]
  t = jnp.maximum(t, 0.0)
  t = jnp.dot(t, w2_ref[...], preferred_element_type=_F32) + b2_ref[...]
  o_ref[...] = jnp.maximum(t, 0.0)


def _layer_mlp(h, agg, p, width):
  row = lambda v: v.reshape(1, -1)
  args = [h, agg, p["l1"]["W"], row(p["l1"]["b"]), row(p["g"]), row(p["be"]),
          p["l2"]["W"], row(p["l2"]["b"])]
  in_specs = [
      pl.BlockSpec((_BLK, width), lambda i: (i, 0)),
      pl.BlockSpec((_BLK, width), lambda i: (i, 0)),
      pl.BlockSpec((width, _H), lambda i: (0, 0)),
      pl.BlockSpec((1, _H), lambda i: (0, 0)),
      pl.BlockSpec((1, _H), lambda i: (0, 0)),
      pl.BlockSpec((1, _H), lambda i: (0, 0)),
      pl.BlockSpec((_H, _H), lambda i: (0, 0)),
      pl.BlockSpec((1, _H), lambda i: (0, 0)),
  ]
  return pl.pallas_call(
      _layer_body,
      grid=(_NBLK,),
      in_specs=in_specs,
      out_specs=pl.BlockSpec((_BLK, _H), lambda i: (i, 0)),
      out_shape=jax.ShapeDtypeStruct((_N, _H), _F32),
  )(*args)


def _emb_body(pooled_ref, lw_ref, lb_ref, o_ref):
  o_ref[...] = jnp.maximum(
      jnp.dot(pooled_ref[...], lw_ref[...], preferred_element_type=_F32)
      + lb_ref[...], 0.0)


def _emb_kernel(pooled, params):
  args = [pooled, params["lin1"]["W"], params["lin1"]["b"].reshape(1, _H)]
  full = lambda a: pl.BlockSpec(a.shape, lambda: tuple(0 for _ in a.shape))
  return pl.pallas_call(
      _emb_body,
      in_specs=[full(a) for a in args],
      out_specs=pl.BlockSpec((_G, _H), lambda: (0, 0)),
      out_shape=jax.ShapeDtypeStruct((_G, _H), _F32),
  )(*args)


def _cdist(a, b):
  sq = (jnp.sum(a * a, axis=1)[:, None] + jnp.sum(b * b, axis=1)[None, :]
        - 2.0 * (a @ b.T))
  return jnp.sqrt(jnp.clip(sq, 0.0, None) + 1e-12)


def _feat_body(emb_ref, oh_ref, c_ref, cdm_ref, sz_ref, w0a_ref, w0b_ref,
               w0c_ref, w0d_ref, b0_ref, w1_ref, b1_ref, w2_ref, b2_ref,
               w3_ref, b3_ref, o_ref):
  kp = 8
  emb = emb_ref[...]
  oh = oh_ref[...]
  col = lax.broadcasted_iota(jnp.int32, (_G, kp), 1)
  validc = jnp.where(col < _K, 1.0, 0.0).astype(_F32)

  ncc = jnp.dot(oh, c_ref[...], preferred_element_type=_F32)
  nto = jnp.dot(oh, cdm_ref[...], preferred_element_type=_F32)
  sizes_sel = jnp.sum(oh * sz_ref[...], axis=1, keepdims=True)
  multi = jnp.sum(
      jnp.where(nto < _THRESH, 1.0, 0.0).astype(_F32) * validc,
      axis=1, keepdims=True)

  ones11 = jnp.ones((1, 1), _F32)

  def _t(v):
    return lax.dot_general(ones11, v, (((0,), (1,)), ((), ())),
                           preferred_element_type=_F32)

  n2e = jnp.sum(emb * emb, axis=1, keepdims=True)
  s_full = lax.dot_general(emb, emb, (((1,), (1,)), ((), ())),
                           preferred_element_type=_F32)
  ndist = jnp.sqrt(jnp.clip(n2e + _t(n2e) - 2.0 * s_full, 0.0, None)
                   + 1e-12)
  nd_mean = jnp.sum(ndist, axis=1, keepdims=True) * _F32(1.0 / _G)

  h2 = (jnp.dot(emb, w0a_ref[...], preferred_element_type=_F32)
        + jnp.dot(ncc, w0b_ref[...], preferred_element_type=_F32)
        + jnp.dot(nto * validc, w0c_ref[...], preferred_element_type=_F32)
        + multi * w0d_ref[0:1]
        + nd_mean * w0d_ref[1:2]
        + sizes_sel * w0d_ref[2:3]
        + b0_ref[...])
  h2 = jnp.maximum(h2, 0.0)
  h2 = jnp.maximum(
      jnp.dot(h2, w1_ref[...], preferred_element_type=_F32) + b1_ref[...],
      0.0)
  h2 = jnp.maximum(
      jnp.dot(h2, w2_ref[...], preferred_element_type=_F32) + b2_ref[...],
      0.0)
  o_ref[...] = (jnp.dot(h2, w3_ref[...], preferred_element_type=_F32)
                + b3_ref[...])


def _final(pooled, params):
  emb = _emb_kernel(pooled, params)

  centers = emb[:_K]
  for _ in range(_KM_ITERS):
    dmat = _cdist(emb, centers)
    labels = jnp.argmin(dmat, axis=1)
    sums = jax.ops.segment_sum(emb, labels, num_segments=_K)
    counts = jax.ops.segment_sum(jnp.ones((_G,), _F32), labels,
                                 num_segments=_K)
    centers = jnp.where(counts[:, None] > 0,
                        sums / jnp.maximum(counts, 1.0)[:, None], centers)
  labels = jnp.argmin(_cdist(emb, centers), axis=1)
  sizes = jax.ops.segment_sum(jnp.ones((_G,), _F32), labels,
                              num_segments=_K)
  cdm = _cdist(centers, centers)

  kp = 8
  onehot = (labels[:, None] == jnp.arange(kp)[None, :]).astype(_F32)
  c_pad = jnp.concatenate([centers, jnp.zeros((1, _H), _F32)], axis=0)
  cdm_pad = jnp.zeros((kp, kp), _F32).at[:_K, :_K].set(cdm)
  sz_pad = jnp.concatenate([sizes, jnp.zeros((1,), _F32)]).reshape(1, kp)

  reg = params["reg"]
  w0 = reg[0]["W"]
  w0a, w0b = w0[0:_H], w0[_H:2 * _H]
  w0c = jnp.concatenate(
      [w0[2 * _H:2 * _H + _K], jnp.zeros((1, 8), _F32)], axis=0)
  w0d = w0[2 * _H + _K:2 * _H + _K + 3]
  args = [emb, onehot, c_pad, cdm_pad, sz_pad,
          w0a, w0b, w0c, w0d, reg[0]["b"].reshape(1, 8),
          reg[1]["W"], reg[1]["b"].reshape(1, 4),
          reg[2]["W"], reg[2]["b"].reshape(1, 2),
          reg[3]["W"], reg[3]["b"].reshape(1, 1)]
  full = lambda a: pl.BlockSpec(a.shape, lambda: tuple(0 for _ in a.shape))
  return pl.pallas_call(
      _feat_body,
      in_specs=[full(a) for a in args],
      out_specs=pl.BlockSpec((_G, 1), lambda: (0, 0)),
      out_shape=jax.ShapeDtypeStruct((_G, 1), _F32),
  )(*args)


def kernel(x, edge_index, batch, params):
  src = edge_index[0]
  dst = edge_index[1]
  convs = params["convs"]

  iota = jnp.arange(_E, dtype=jnp.int32)
  sd, perm = lax.sort([dst, iota], num_keys=1, is_stable=True)
  ss = src[perm]

  h = x
  width = _D
  for p in convs:
    agg = _seg_sum_sorted(h, ss, sd, _ACCN, width)
    h = _layer_mlp(h, agg, p, width)
    width = _H

  pool_ss = jnp.arange(_N, dtype=jnp.int32)
  pooled = _seg_sum_sorted(h, pool_ss, batch, _G, _H)
  return _final(pooled, params)


# speedup vs baseline: 3.1941x; 1.2080x over previous
import functools

import numpy as np
import jax
import jax.numpy as jnp
from jax import lax
from jax.experimental import pallas as pl
from jax.experimental.pallas import tpu as pltpu
from jax.experimental.pallas import tpu_sc as plsc

_N, _E, _D, _H, _G, _K = 10000, 320000, 128, 32, 512, 7
_KM_ITERS = 10
_THRESH = 1.0
_BN_DEN = float(np.sqrt(np.float32(1.0 + 1e-5), dtype=np.float32))

_NC, _NS = 2, 16
_NW = _NC * _NS
_CH = 80

_F32 = jnp.float32


@functools.cache
def _make_seg_sum(acc_rows, width, epad):
  rpw = acc_rows // _NW
  assert rpw % 8 == 0 and epad % 8 == 0

  mesh = plsc.VectorSubcoreMesh(core_axis_name="c", subcore_axis_name="s")

  @functools.partial(
      pl.kernel,
      out_type=jax.ShapeDtypeStruct((acc_rows, width), _F32),
      mesh=mesh,
      scratch_types=[
          pltpu.VMEM((_CH,), jnp.int32),
          pltpu.VMEM((_CH,), jnp.int32),
          pltpu.VMEM((_CH, width), _F32),
          pltpu.VMEM((_CH,), jnp.int32),
          pltpu.VMEM((_CH,), jnp.int32),
          pltpu.VMEM((_CH, width), _F32),
          pltpu.VMEM((80,), jnp.int32),
          pltpu.VMEM((rpw + 8, width), _F32),
          pltpu.SemaphoreType.DMA,
          pltpu.SemaphoreType.DMA,
      ],
      compiler_params=pltpu.CompilerParams(use_tc_tiling_on_sc=False),
  )
  def seg_sum(vals_hbm, ss_hbm, sd_hbm, meta_hbm, out_hbm,
              ssv0, sdv0, rows0, ssv1, sdv1, rows1, meta_v, acc,
              sem0, sem1):
    c = lax.axis_index("c")
    s = lax.axis_index("s")
    w = c * _NS + s
    lo = w * rpw

    pltpu.sync_copy(meta_hbm, meta_v)
    astart = meta_v[pl.ds(w, 16)][0]
    nch = meta_v[pl.ds(32 + w, 16)][0]

    def zero_row(i, carry):
      for j in range(width // 16):
        acc[i, pl.ds(16 * j, 16)] = jnp.zeros((16,), _F32)
      return carry

    lax.fori_loop(0, rpw + 8, zero_row, 0)

    def load_idx(i, ssv, sdv):
      off = pl.multiple_of(astart + i * _CH, 8)
      pltpu.sync_copy(ss_hbm.at[pl.ds(off, _CH)], ssv)
      pltpu.sync_copy(sd_hbm.at[pl.ds(off, _CH)], sdv)

    def fire(ssv, rows, sem):
      pltpu.async_copy(vals_hbm.at[ssv], rows, sem)

    def drain(ssv, rows, sem):
      pltpu.make_async_copy(vals_hbm.at[ssv], rows, sem).wait()

    def accum(sdv, rows):
      def group(g, carry2):
        dvec = sdv[pl.ds(g * 16, 16)] - lo
        okv = jnp.logical_and(dvec >= 0, dvec < rpw)
        dlv = jnp.where(okv, dvec, rpw)
        for j in range(16):
          dl = dlv[j]
          e = g * 16 + j
          for k in range(width // 16):
            acc[dl, pl.ds(16 * k, 16)] = (
                acc[dl, pl.ds(16 * k, 16)] + rows[e, pl.ds(16 * k, 16)])
        return carry2

      lax.fori_loop(0, _CH // 16, group, 0)

    @pl.when(nch > 0)
    def _():
      load_idx(0, ssv0, sdv0)
      fire(ssv0, rows0, sem0)

    def pair(i, carry):
      c0 = 2 * i

      @pl.when(c0 + 1 < nch)
      def _():
        load_idx(c0 + 1, ssv1, sdv1)
        fire(ssv1, rows1, sem1)

      drain(ssv0, rows0, sem0)
      accum(sdv0, rows0)

      @pl.when(c0 + 2 < nch)
      def _():
        load_idx(c0 + 2, ssv0, sdv0)
        fire(ssv0, rows0, sem0)

      @pl.when(c0 + 1 < nch)
      def _():
        drain(ssv1, rows1, sem1)
        accum(sdv1, rows1)

      return carry

    lax.fori_loop(0, (nch + 1) // 2, pair, 0)
    pltpu.sync_copy(acc.at[pl.ds(0, rpw)], out_hbm.at[pl.ds(lo, rpw)])

  return seg_sum


_ACCN = 10240


def _seg_sum_sorted(vals, ss_sorted, sd_sorted, acc_rows, width):
  n_idx = sd_sorted.shape[0]
  ss_p = jnp.concatenate(
      [ss_sorted, jnp.zeros((_CH,), jnp.int32)])
  sd_p = jnp.concatenate(
      [sd_sorted, jnp.full((_CH,), acc_rows, jnp.int32)])
  rpw = acc_rows // _NW
  starts = jnp.searchsorted(
      sd_sorted, jnp.arange(33, dtype=jnp.int32) * rpw).astype(jnp.int32)
  astart = (starts[:32] // 8) * 8
  aend = jnp.minimum(((starts[1:] + 7) // 8) * 8, n_idx)
  nch = jnp.maximum((aend - astart + _CH - 1) // _CH, 0).astype(jnp.int32)
  meta = jnp.concatenate(
      [astart, nch, jnp.zeros((16,), jnp.int32)]).astype(jnp.int32)
  return _make_seg_sum(acc_rows, width, n_idx + _CH)(vals, ss_p, sd_p, meta)


_BLK = 1000
_NBLK = _N // _BLK


def _layer_body(h_ref, agg_ref, w1_ref, b1_ref, g_ref, be_ref, w2_ref,
                b2_ref, o_ref):
  hs = h_ref[...] + agg_ref[...]
  t = jnp.dot(hs, w1_ref[...], preferred_element_type=_F32) + b1_ref[...]
  t = g_ref[...] * t / _BN_DEN + be_ref[...]
  t = jnp.maximum(t, 0.0)
  t = jnp.dot(t, w2_ref[...], preferred_element_type=_F32) + b2_ref[...]
  o_ref[...] = jnp.maximum(t, 0.0)


def _layer_mlp(h, agg, p, width):
  row = lambda v: v.reshape(1, -1)
  args = [h, agg, p["l1"]["W"], row(p["l1"]["b"]), row(p["g"]), row(p["be"]),
          p["l2"]["W"], row(p["l2"]["b"])]
  in_specs = [
      pl.BlockSpec((_BLK, width), lambda i: (i, 0)),
      pl.BlockSpec((_BLK, width), lambda i: (i, 0)),
      pl.BlockSpec((width, _H), lambda i: (0, 0)),
      pl.BlockSpec((1, _H), lambda i: (0, 0)),
      pl.BlockSpec((1, _H), lambda i: (0, 0)),
      pl.BlockSpec((1, _H), lambda i: (0, 0)),
      pl.BlockSpec((_H, _H), lambda i: (0, 0)),
      pl.BlockSpec((1, _H), lambda i: (0, 0)),
  ]
  return pl.pallas_call(
      _layer_body,
      grid=(_NBLK,),
      in_specs=in_specs,
      out_specs=pl.BlockSpec((_BLK, _H), lambda i: (i, 0)),
      out_shape=jax.ShapeDtypeStruct((_N, _H), _F32),
  )(*args)


def _emb_body(pooled_ref, lw_ref, lb_ref, o_ref):
  o_ref[...] = jnp.maximum(
      jnp.dot(pooled_ref[...], lw_ref[...], preferred_element_type=_F32)
      + lb_ref[...], 0.0)


def _emb_kernel(pooled, params):
  args = [pooled, params["lin1"]["W"], params["lin1"]["b"].reshape(1, _H)]
  full = lambda a: pl.BlockSpec(a.shape, lambda: tuple(0 for _ in a.shape))
  return pl.pallas_call(
      _emb_body,
      in_specs=[full(a) for a in args],
      out_specs=pl.BlockSpec((_G, _H), lambda: (0, 0)),
      out_shape=jax.ShapeDtypeStruct((_G, _H), _F32),
  )(*args)


def _cdist(a, b):
  sq = (jnp.sum(a * a, axis=1)[:, None] + jnp.sum(b * b, axis=1)[None, :]
        - 2.0 * (a @ b.T))
  return jnp.sqrt(jnp.clip(sq, 0.0, None) + 1e-12)


def _feat_body(emb_ref, oh_ref, c_ref, cdm_ref, sz_ref, w0a_ref, w0b_ref,
               w0c_ref, w0d_ref, b0_ref, w1_ref, b1_ref, w2_ref, b2_ref,
               w3_ref, b3_ref, o_ref):
  kp = 8
  emb = emb_ref[...]
  oh = oh_ref[...]
  col = lax.broadcasted_iota(jnp.int32, (_G, kp), 1)
  validc = jnp.where(col < _K, 1.0, 0.0).astype(_F32)

  ncc = jnp.dot(oh, c_ref[...], preferred_element_type=_F32)
  nto = jnp.dot(oh, cdm_ref[...], preferred_element_type=_F32)
  sizes_sel = jnp.sum(oh * sz_ref[...], axis=1, keepdims=True)
  multi = jnp.sum(
      jnp.where(nto < _THRESH, 1.0, 0.0).astype(_F32) * validc,
      axis=1, keepdims=True)

  ones11 = jnp.ones((1, 1), _F32)

  def _t(v):
    return lax.dot_general(ones11, v, (((0,), (1,)), ((), ())),
                           preferred_element_type=_F32)

  n2e = jnp.sum(emb * emb, axis=1, keepdims=True)
  s_full = lax.dot_general(emb, emb, (((1,), (1,)), ((), ())),
                           preferred_element_type=_F32)
  ndist = jnp.sqrt(jnp.clip(n2e + _t(n2e) - 2.0 * s_full, 0.0, None)
                   + 1e-12)
  nd_mean = jnp.sum(ndist, axis=1, keepdims=True) * _F32(1.0 / _G)

  h2 = (jnp.dot(emb, w0a_ref[...], preferred_element_type=_F32)
        + jnp.dot(ncc, w0b_ref[...], preferred_element_type=_F32)
        + jnp.dot(nto * validc, w0c_ref[...], preferred_element_type=_F32)
        + multi * w0d_ref[0:1]
        + nd_mean * w0d_ref[1:2]
        + sizes_sel * w0d_ref[2:3]
        + b0_ref[...])
  h2 = jnp.maximum(h2, 0.0)
  h2 = jnp.maximum(
      jnp.dot(h2, w1_ref[...], preferred_element_type=_F32) + b1_ref[...],
      0.0)
  h2 = jnp.maximum(
      jnp.dot(h2, w2_ref[...], preferred_element_type=_F32) + b2_ref[...],
      0.0)
  o_ref[...] = (jnp.dot(h2, w3_ref[...], preferred_element_type=_F32)
                + b3_ref[...])


def _final(pooled, params):
  emb = _emb_kernel(pooled, params)

  centers = emb[:_K]
  for _ in range(_KM_ITERS):
    dmat = _cdist(emb, centers)
    labels = jnp.argmin(dmat, axis=1)
    sums = jax.ops.segment_sum(emb, labels, num_segments=_K)
    counts = jax.ops.segment_sum(jnp.ones((_G,), _F32), labels,
                                 num_segments=_K)
    centers = jnp.where(counts[:, None] > 0,
                        sums / jnp.maximum(counts, 1.0)[:, None], centers)
  labels = jnp.argmin(_cdist(emb, centers), axis=1)
  sizes = jax.ops.segment_sum(jnp.ones((_G,), _F32), labels,
                              num_segments=_K)
  cdm = _cdist(centers, centers)

  kp = 8
  onehot = (labels[:, None] == jnp.arange(kp)[None, :]).astype(_F32)
  c_pad = jnp.concatenate([centers, jnp.zeros((1, _H), _F32)], axis=0)
  cdm_pad = jnp.zeros((kp, kp), _F32).at[:_K, :_K].set(cdm)
  sz_pad = jnp.concatenate([sizes, jnp.zeros((1,), _F32)]).reshape(1, kp)

  reg = params["reg"]
  w0 = reg[0]["W"]
  w0a, w0b = w0[0:_H], w0[_H:2 * _H]
  w0c = jnp.concatenate(
      [w0[2 * _H:2 * _H + _K], jnp.zeros((1, 8), _F32)], axis=0)
  w0d = w0[2 * _H + _K:2 * _H + _K + 3]
  args = [emb, onehot, c_pad, cdm_pad, sz_pad,
          w0a, w0b, w0c, w0d, reg[0]["b"].reshape(1, 8),
          reg[1]["W"], reg[1]["b"].reshape(1, 4),
          reg[2]["W"], reg[2]["b"].reshape(1, 2),
          reg[3]["W"], reg[3]["b"].reshape(1, 1)]
  full = lambda a: pl.BlockSpec(a.shape, lambda: tuple(0 for _ in a.shape))
  return pl.pallas_call(
      _feat_body,
      in_specs=[full(a) for a in args],
      out_specs=pl.BlockSpec((_G, 1), lambda: (0, 0)),
      out_shape=jax.ShapeDtypeStruct((_G, 1), _F32),
  )(*args)


def kernel(x, edge_index, batch, params):
  src = edge_index[0]
  dst = edge_index[1]
  convs = params["convs"]

  iota = jnp.arange(_E, dtype=jnp.int32)
  sd, perm = lax.sort([dst, iota], num_keys=1, is_stable=True)
  ss = src[perm]

  h = x
  width = _D
  for p in convs:
    agg = _seg_sum_sorted(h, ss, sd, _ACCN, width)
    h = _layer_mlp(h, agg, p, width)
    width = _H

  pool_ss = jnp.arange(_N, dtype=jnp.int32)
  pooled = _seg_sum_sorted(h, pool_ss, batch, _G, _H)
  return _final(pooled, params)


# speedup vs baseline: 3.5699x; 1.1177x over previous
import functools

import numpy as np
import jax
import jax.numpy as jnp
from jax import lax
from jax.experimental import pallas as pl
from jax.experimental.pallas import tpu as pltpu
from jax.experimental.pallas import tpu_sc as plsc

_N, _E, _D, _H, _G, _K = 10000, 320000, 128, 32, 512, 7
_KM_ITERS = 10
_THRESH = 1.0
_BN_DEN = float(np.sqrt(np.float32(1.0 + 1e-5), dtype=np.float32))

_NC, _NS = 2, 16
_NW = _NC * _NS
_CH = 160

_F32 = jnp.float32


@functools.cache
def _make_seg_sum(acc_rows, width, epad):
  rpw = acc_rows // _NW
  assert rpw % 8 == 0 and epad % 8 == 0

  mesh = plsc.VectorSubcoreMesh(core_axis_name="c", subcore_axis_name="s")

  @functools.partial(
      pl.kernel,
      out_type=jax.ShapeDtypeStruct((acc_rows, width), _F32),
      mesh=mesh,
      scratch_types=[
          pltpu.VMEM((_CH,), jnp.int32),
          pltpu.VMEM((_CH,), jnp.int32),
          pltpu.VMEM((_CH, width), _F32),
          pltpu.VMEM((_CH,), jnp.int32),
          pltpu.VMEM((_CH,), jnp.int32),
          pltpu.VMEM((_CH, width), _F32),
          pltpu.VMEM((80,), jnp.int32),
          pltpu.VMEM((rpw + 8, width), _F32),
          pltpu.SemaphoreType.DMA,
          pltpu.SemaphoreType.DMA,
      ],
      compiler_params=pltpu.CompilerParams(use_tc_tiling_on_sc=False),
  )
  def seg_sum(vals_hbm, ss_hbm, sd_hbm, meta_hbm, out_hbm,
              ssv0, sdv0, rows0, ssv1, sdv1, rows1, meta_v, acc,
              sem0, sem1):
    c = lax.axis_index("c")
    s = lax.axis_index("s")
    w = c * _NS + s
    lo = w * rpw

    pltpu.sync_copy(meta_hbm, meta_v)
    astart = meta_v[pl.ds(w, 16)][0]
    nch = meta_v[pl.ds(32 + w, 16)][0]

    def zero_row(i, carry):
      for j in range(width // 16):
        acc[i, pl.ds(16 * j, 16)] = jnp.zeros((16,), _F32)
      return carry

    lax.fori_loop(0, rpw + 8, zero_row, 0)

    def load_idx(i, ssv, sdv):
      off = pl.multiple_of(astart + i * _CH, 8)
      pltpu.sync_copy(ss_hbm.at[pl.ds(off, _CH)], ssv)
      pltpu.sync_copy(sd_hbm.at[pl.ds(off, _CH)], sdv)

    def fire(ssv, rows, sem):
      pltpu.async_copy(vals_hbm.at[ssv], rows, sem)

    def drain(ssv, rows, sem):
      pltpu.make_async_copy(vals_hbm.at[ssv], rows, sem).wait()

    def accum(sdv, rows):
      def group(g, carry2):
        dvec = sdv[pl.ds(g * 16, 16)] - lo
        okv = jnp.logical_and(dvec >= 0, dvec < rpw)
        dlv = jnp.where(okv, dvec, rpw)
        for j in range(16):
          dl = dlv[j]
          e = g * 16 + j
          for k in range(width // 16):
            acc[dl, pl.ds(16 * k, 16)] = (
                acc[dl, pl.ds(16 * k, 16)] + rows[e, pl.ds(16 * k, 16)])
        return carry2

      lax.fori_loop(0, _CH // 16, group, 0)

    @pl.when(nch > 0)
    def _():
      load_idx(0, ssv0, sdv0)
      fire(ssv0, rows0, sem0)

    def pair(i, carry):
      c0 = 2 * i

      @pl.when(c0 + 1 < nch)
      def _():
        load_idx(c0 + 1, ssv1, sdv1)
        fire(ssv1, rows1, sem1)

      drain(ssv0, rows0, sem0)
      accum(sdv0, rows0)

      @pl.when(c0 + 2 < nch)
      def _():
        load_idx(c0 + 2, ssv0, sdv0)
        fire(ssv0, rows0, sem0)

      @pl.when(c0 + 1 < nch)
      def _():
        drain(ssv1, rows1, sem1)
        accum(sdv1, rows1)

      return carry

    lax.fori_loop(0, (nch + 1) // 2, pair, 0)
    pltpu.sync_copy(acc.at[pl.ds(0, rpw)], out_hbm.at[pl.ds(lo, rpw)])

  return seg_sum


_ACCN = 10240


def _seg_sum_sorted(vals, ss_sorted, sd_sorted, acc_rows, width):
  n_idx = sd_sorted.shape[0]
  ss_p = jnp.concatenate(
      [ss_sorted, jnp.zeros((_CH,), jnp.int32)])
  sd_p = jnp.concatenate(
      [sd_sorted, jnp.full((_CH,), acc_rows, jnp.int32)])
  rpw = acc_rows // _NW
  starts = jnp.searchsorted(
      sd_sorted, jnp.arange(33, dtype=jnp.int32) * rpw).astype(jnp.int32)
  astart = (starts[:32] // 8) * 8
  aend = jnp.minimum(((starts[1:] + 7) // 8) * 8, n_idx)
  nch = jnp.maximum((aend - astart + _CH - 1) // _CH, 0).astype(jnp.int32)
  meta = jnp.concatenate(
      [astart, nch, jnp.zeros((16,), jnp.int32)]).astype(jnp.int32)
  return _make_seg_sum(acc_rows, width, n_idx + _CH)(vals, ss_p, sd_p, meta)


_BLK = 1000
_NBLK = _N // _BLK


def _layer_body(h_ref, agg_ref, w1_ref, b1_ref, g_ref, be_ref, w2_ref,
                b2_ref, o_ref):
  hs = h_ref[...] + agg_ref[...]
  t = jnp.dot(hs, w1_ref[...], preferred_element_type=_F32) + b1_ref[...]
  t = g_ref[...] * t / _BN_DEN + be_ref[...]
  t = jnp.maximum(t, 0.0)
  t = jnp.dot(t, w2_ref[...], preferred_element_type=_F32) + b2_ref[...]
  o_ref[...] = jnp.maximum(t, 0.0)


def _layer_mlp(h, agg, p, width):
  row = lambda v: v.reshape(1, -1)
  args = [h, agg, p["l1"]["W"], row(p["l1"]["b"]), row(p["g"]), row(p["be"]),
          p["l2"]["W"], row(p["l2"]["b"])]
  in_specs = [
      pl.BlockSpec((_BLK, width), lambda i: (i, 0)),
      pl.BlockSpec((_BLK, width), lambda i: (i, 0)),
      pl.BlockSpec((width, _H), lambda i: (0, 0)),
      pl.BlockSpec((1, _H), lambda i: (0, 0)),
      pl.BlockSpec((1, _H), lambda i: (0, 0)),
      pl.BlockSpec((1, _H), lambda i: (0, 0)),
      pl.BlockSpec((_H, _H), lambda i: (0, 0)),
      pl.BlockSpec((1, _H), lambda i: (0, 0)),
  ]
  return pl.pallas_call(
      _layer_body,
      grid=(_NBLK,),
      in_specs=in_specs,
      out_specs=pl.BlockSpec((_BLK, _H), lambda i: (i, 0)),
      out_shape=jax.ShapeDtypeStruct((_N, _H), _F32),
  )(*args)


def _emb_body(pooled_ref, lw_ref, lb_ref, o_ref):
  o_ref[...] = jnp.maximum(
      jnp.dot(pooled_ref[...], lw_ref[...], preferred_element_type=_F32)
      + lb_ref[...], 0.0)


def _emb_kernel(pooled, params):
  args = [pooled, params["lin1"]["W"], params["lin1"]["b"].reshape(1, _H)]
  full = lambda a: pl.BlockSpec(a.shape, lambda: tuple(0 for _ in a.shape))
  return pl.pallas_call(
      _emb_body,
      in_specs=[full(a) for a in args],
      out_specs=pl.BlockSpec((_G, _H), lambda: (0, 0)),
      out_shape=jax.ShapeDtypeStruct((_G, _H), _F32),
  )(*args)


def _cdist(a, b):
  sq = (jnp.sum(a * a, axis=1)[:, None] + jnp.sum(b * b, axis=1)[None, :]
        - 2.0 * (a @ b.T))
  return jnp.sqrt(jnp.clip(sq, 0.0, None) + 1e-12)


def _feat_body(emb_ref, oh_ref, c_ref, cdm_ref, sz_ref, w0a_ref, w0b_ref,
               w0c_ref, w0d_ref, b0_ref, w1_ref, b1_ref, w2_ref, b2_ref,
               w3_ref, b3_ref, o_ref):
  kp = 8
  emb = emb_ref[...]
  oh = oh_ref[...]
  col = lax.broadcasted_iota(jnp.int32, (_G, kp), 1)
  validc = jnp.where(col < _K, 1.0, 0.0).astype(_F32)

  ncc = jnp.dot(oh, c_ref[...], preferred_element_type=_F32)
  nto = jnp.dot(oh, cdm_ref[...], preferred_element_type=_F32)
  sizes_sel = jnp.sum(oh * sz_ref[...], axis=1, keepdims=True)
  multi = jnp.sum(
      jnp.where(nto < _THRESH, 1.0, 0.0).astype(_F32) * validc,
      axis=1, keepdims=True)

  ones11 = jnp.ones((1, 1), _F32)

  def _t(v):
    return lax.dot_general(ones11, v, (((0,), (1,)), ((), ())),
                           preferred_element_type=_F32)

  n2e = jnp.sum(emb * emb, axis=1, keepdims=True)
  s_full = lax.dot_general(emb, emb, (((1,), (1,)), ((), ())),
                           preferred_element_type=_F32)
  ndist = jnp.sqrt(jnp.clip(n2e + _t(n2e) - 2.0 * s_full, 0.0, None)
                   + 1e-12)
  nd_mean = jnp.sum(ndist, axis=1, keepdims=True) * _F32(1.0 / _G)

  h2 = (jnp.dot(emb, w0a_ref[...], preferred_element_type=_F32)
        + jnp.dot(ncc, w0b_ref[...], preferred_element_type=_F32)
        + jnp.dot(nto * validc, w0c_ref[...], preferred_element_type=_F32)
        + multi * w0d_ref[0:1]
        + nd_mean * w0d_ref[1:2]
        + sizes_sel * w0d_ref[2:3]
        + b0_ref[...])
  h2 = jnp.maximum(h2, 0.0)
  h2 = jnp.maximum(
      jnp.dot(h2, w1_ref[...], preferred_element_type=_F32) + b1_ref[...],
      0.0)
  h2 = jnp.maximum(
      jnp.dot(h2, w2_ref[...], preferred_element_type=_F32) + b2_ref[...],
      0.0)
  o_ref[...] = (jnp.dot(h2, w3_ref[...], preferred_element_type=_F32)
                + b3_ref[...])


def _final(pooled, params):
  emb = _emb_kernel(pooled, params)

  centers = emb[:_K]
  for _ in range(_KM_ITERS):
    dmat = _cdist(emb, centers)
    labels = jnp.argmin(dmat, axis=1)
    sums = jax.ops.segment_sum(emb, labels, num_segments=_K)
    counts = jax.ops.segment_sum(jnp.ones((_G,), _F32), labels,
                                 num_segments=_K)
    centers = jnp.where(counts[:, None] > 0,
                        sums / jnp.maximum(counts, 1.0)[:, None], centers)
  labels = jnp.argmin(_cdist(emb, centers), axis=1)
  sizes = jax.ops.segment_sum(jnp.ones((_G,), _F32), labels,
                              num_segments=_K)
  cdm = _cdist(centers, centers)

  kp = 8
  onehot = (labels[:, None] == jnp.arange(kp)[None, :]).astype(_F32)
  c_pad = jnp.concatenate([centers, jnp.zeros((1, _H), _F32)], axis=0)
  cdm_pad = jnp.zeros((kp, kp), _F32).at[:_K, :_K].set(cdm)
  sz_pad = jnp.concatenate([sizes, jnp.zeros((1,), _F32)]).reshape(1, kp)

  reg = params["reg"]
  w0 = reg[0]["W"]
  w0a, w0b = w0[0:_H], w0[_H:2 * _H]
  w0c = jnp.concatenate(
      [w0[2 * _H:2 * _H + _K], jnp.zeros((1, 8), _F32)], axis=0)
  w0d = w0[2 * _H + _K:2 * _H + _K + 3]
  args = [emb, onehot, c_pad, cdm_pad, sz_pad,
          w0a, w0b, w0c, w0d, reg[0]["b"].reshape(1, 8),
          reg[1]["W"], reg[1]["b"].reshape(1, 4),
          reg[2]["W"], reg[2]["b"].reshape(1, 2),
          reg[3]["W"], reg[3]["b"].reshape(1, 1)]
  full = lambda a: pl.BlockSpec(a.shape, lambda: tuple(0 for _ in a.shape))
  return pl.pallas_call(
      _feat_body,
      in_specs=[full(a) for a in args],
      out_specs=pl.BlockSpec((_G, 1), lambda: (0, 0)),
      out_shape=jax.ShapeDtypeStruct((_G, 1), _F32),
  )(*args)


def kernel(x, edge_index, batch, params):
  src = edge_index[0]
  dst = edge_index[1]
  convs = params["convs"]

  iota = jnp.arange(_E, dtype=jnp.int32)
  sd, perm = lax.sort([dst, iota], num_keys=1, is_stable=True)
  ss = src[perm]

  h = x
  width = _D
  for p in convs:
    agg = _seg_sum_sorted(h, ss, sd, _ACCN, width)
    h = _layer_mlp(h, agg, p, width)
    width = _H

  pool_ss = jnp.arange(_N, dtype=jnp.int32)
  pooled = _seg_sum_sorted(h, pool_ss, batch, _G, _H)
  return _final(pooled, params)


# speedup vs baseline: 3.7232x; 1.0429x over previous
import functools

import numpy as np
import jax
import jax.numpy as jnp
from jax import lax
from jax.experimental import pallas as pl
from jax.experimental.pallas import tpu as pltpu
from jax.experimental.pallas import tpu_sc as plsc

_N, _E, _D, _H, _G, _K = 10000, 320000, 128, 32, 512, 7
_KM_ITERS = 10
_THRESH = 1.0
_BN_DEN = float(np.sqrt(np.float32(1.0 + 1e-5), dtype=np.float32))

_NC, _NS = 2, 16
_NW = _NC * _NS
_CH = 240

_F32 = jnp.float32


@functools.cache
def _make_seg_sum(acc_rows, width, epad):
  rpw = acc_rows // _NW
  assert rpw % 8 == 0 and epad % 8 == 0

  mesh = plsc.VectorSubcoreMesh(core_axis_name="c", subcore_axis_name="s")

  @functools.partial(
      pl.kernel,
      out_type=jax.ShapeDtypeStruct((acc_rows, width), _F32),
      mesh=mesh,
      scratch_types=[
          pltpu.VMEM((_CH,), jnp.int32),
          pltpu.VMEM((_CH,), jnp.int32),
          pltpu.VMEM((_CH, width), _F32),
          pltpu.VMEM((_CH,), jnp.int32),
          pltpu.VMEM((_CH,), jnp.int32),
          pltpu.VMEM((_CH, width), _F32),
          pltpu.VMEM((80,), jnp.int32),
          pltpu.VMEM((rpw + 8, width), _F32),
          pltpu.SemaphoreType.DMA,
          pltpu.SemaphoreType.DMA,
      ],
      compiler_params=pltpu.CompilerParams(use_tc_tiling_on_sc=False),
  )
  def seg_sum(vals_hbm, ss_hbm, sd_hbm, meta_hbm, out_hbm,
              ssv0, sdv0, rows0, ssv1, sdv1, rows1, meta_v, acc,
              sem0, sem1):
    c = lax.axis_index("c")
    s = lax.axis_index("s")
    w = c * _NS + s
    lo = w * rpw

    pltpu.sync_copy(meta_hbm, meta_v)
    astart = meta_v[pl.ds(w, 16)][0]
    nch = meta_v[pl.ds(32 + w, 16)][0]

    def zero_row(i, carry):
      for j in range(width // 16):
        acc[i, pl.ds(16 * j, 16)] = jnp.zeros((16,), _F32)
      return carry

    lax.fori_loop(0, rpw + 8, zero_row, 0)

    def load_idx(i, ssv, sdv):
      off = pl.multiple_of(astart + i * _CH, 8)
      pltpu.sync_copy(ss_hbm.at[pl.ds(off, _CH)], ssv)
      pltpu.sync_copy(sd_hbm.at[pl.ds(off, _CH)], sdv)

    def fire(ssv, rows, sem):
      pltpu.async_copy(vals_hbm.at[ssv], rows, sem)

    def drain(ssv, rows, sem):
      pltpu.make_async_copy(vals_hbm.at[ssv], rows, sem).wait()

    def accum(sdv, rows):
      def group(g, carry2):
        dvec = sdv[pl.ds(g * 16, 16)] - lo
        okv = jnp.logical_and(dvec >= 0, dvec < rpw)
        dlv = jnp.where(okv, dvec, rpw)
        for j in range(16):
          dl = dlv[j]
          e = g * 16 + j
          for k in range(width // 16):
            acc[dl, pl.ds(16 * k, 16)] = (
                acc[dl, pl.ds(16 * k, 16)] + rows[e, pl.ds(16 * k, 16)])
        return carry2

      lax.fori_loop(0, _CH // 16, group, 0)

    @pl.when(nch > 0)
    def _():
      load_idx(0, ssv0, sdv0)
      fire(ssv0, rows0, sem0)

    def pair(i, carry):
      c0 = 2 * i

      @pl.when(c0 + 1 < nch)
      def _():
        load_idx(c0 + 1, ssv1, sdv1)
        fire(ssv1, rows1, sem1)

      drain(ssv0, rows0, sem0)
      accum(sdv0, rows0)

      @pl.when(c0 + 2 < nch)
      def _():
        load_idx(c0 + 2, ssv0, sdv0)
        fire(ssv0, rows0, sem0)

      @pl.when(c0 + 1 < nch)
      def _():
        drain(ssv1, rows1, sem1)
        accum(sdv1, rows1)

      return carry

    lax.fori_loop(0, (nch + 1) // 2, pair, 0)
    pltpu.sync_copy(acc.at[pl.ds(0, rpw)], out_hbm.at[pl.ds(lo, rpw)])

  return seg_sum


_ACCN = 10240


def _seg_sum_sorted(vals, ss_sorted, sd_sorted, acc_rows, width):
  n_idx = sd_sorted.shape[0]
  ss_p = jnp.concatenate(
      [ss_sorted, jnp.zeros((_CH,), jnp.int32)])
  sd_p = jnp.concatenate(
      [sd_sorted, jnp.full((_CH,), acc_rows, jnp.int32)])
  rpw = acc_rows // _NW
  starts = jnp.searchsorted(
      sd_sorted, jnp.arange(33, dtype=jnp.int32) * rpw).astype(jnp.int32)
  astart = (starts[:32] // 8) * 8
  aend = jnp.minimum(((starts[1:] + 7) // 8) * 8, n_idx)
  nch = jnp.maximum((aend - astart + _CH - 1) // _CH, 0).astype(jnp.int32)
  meta = jnp.concatenate(
      [astart, nch, jnp.zeros((16,), jnp.int32)]).astype(jnp.int32)
  return _make_seg_sum(acc_rows, width, n_idx + _CH)(vals, ss_p, sd_p, meta)


_BLK = 1000
_NBLK = _N // _BLK


def _layer_body(h_ref, agg_ref, w1_ref, b1_ref, g_ref, be_ref, w2_ref,
                b2_ref, o_ref):
  hs = h_ref[...] + agg_ref[...]
  t = jnp.dot(hs, w1_ref[...], preferred_element_type=_F32) + b1_ref[...]
  t = g_ref[...] * t / _BN_DEN + be_ref[...]
  t = jnp.maximum(t, 0.0)
  t = jnp.dot(t, w2_ref[...], preferred_element_type=_F32) + b2_ref[...]
  o_ref[...] = jnp.maximum(t, 0.0)


def _layer_mlp(h, agg, p, width):
  row = lambda v: v.reshape(1, -1)
  args = [h, agg, p["l1"]["W"], row(p["l1"]["b"]), row(p["g"]), row(p["be"]),
          p["l2"]["W"], row(p["l2"]["b"])]
  in_specs = [
      pl.BlockSpec((_BLK, width), lambda i: (i, 0)),
      pl.BlockSpec((_BLK, width), lambda i: (i, 0)),
      pl.BlockSpec((width, _H), lambda i: (0, 0)),
      pl.BlockSpec((1, _H), lambda i: (0, 0)),
      pl.BlockSpec((1, _H), lambda i: (0, 0)),
      pl.BlockSpec((1, _H), lambda i: (0, 0)),
      pl.BlockSpec((_H, _H), lambda i: (0, 0)),
      pl.BlockSpec((1, _H), lambda i: (0, 0)),
  ]
  return pl.pallas_call(
      _layer_body,
      grid=(_NBLK,),
      in_specs=in_specs,
      out_specs=pl.BlockSpec((_BLK, _H), lambda i: (i, 0)),
      out_shape=jax.ShapeDtypeStruct((_N, _H), _F32),
  )(*args)


def _emb_body(pooled_ref, lw_ref, lb_ref, o_ref):
  o_ref[...] = jnp.maximum(
      jnp.dot(pooled_ref[...], lw_ref[...], preferred_element_type=_F32)
      + lb_ref[...], 0.0)


def _emb_kernel(pooled, params):
  args = [pooled, params["lin1"]["W"], params["lin1"]["b"].reshape(1, _H)]
  full = lambda a: pl.BlockSpec(a.shape, lambda: tuple(0 for _ in a.shape))
  return pl.pallas_call(
      _emb_body,
      in_specs=[full(a) for a in args],
      out_specs=pl.BlockSpec((_G, _H), lambda: (0, 0)),
      out_shape=jax.ShapeDtypeStruct((_G, _H), _F32),
  )(*args)


def _cdist(a, b):
  sq = (jnp.sum(a * a, axis=1)[:, None] + jnp.sum(b * b, axis=1)[None, :]
        - 2.0 * (a @ b.T))
  return jnp.sqrt(jnp.clip(sq, 0.0, None) + 1e-12)


def _feat_body(emb_ref, oh_ref, c_ref, cdm_ref, sz_ref, w0a_ref, w0b_ref,
               w0c_ref, w0d_ref, b0_ref, w1_ref, b1_ref, w2_ref, b2_ref,
               w3_ref, b3_ref, o_ref):
  kp = 8
  emb = emb_ref[...]
  oh = oh_ref[...]
  col = lax.broadcasted_iota(jnp.int32, (_G, kp), 1)
  validc = jnp.where(col < _K, 1.0, 0.0).astype(_F32)

  ncc = jnp.dot(oh, c_ref[...], preferred_element_type=_F32)
  nto = jnp.dot(oh, cdm_ref[...], preferred_element_type=_F32)
  sizes_sel = jnp.sum(oh * sz_ref[...], axis=1, keepdims=True)
  multi = jnp.sum(
      jnp.where(nto < _THRESH, 1.0, 0.0).astype(_F32) * validc,
      axis=1, keepdims=True)

  ones11 = jnp.ones((1, 1), _F32)

  def _t(v):
    return lax.dot_general(ones11, v, (((0,), (1,)), ((), ())),
                           preferred_element_type=_F32)

  n2e = jnp.sum(emb * emb, axis=1, keepdims=True)
  s_full = lax.dot_general(emb, emb, (((1,), (1,)), ((), ())),
                           preferred_element_type=_F32)
  ndist = jnp.sqrt(jnp.clip(n2e + _t(n2e) - 2.0 * s_full, 0.0, None)
                   + 1e-12)
  nd_mean = jnp.sum(ndist, axis=1, keepdims=True) * _F32(1.0 / _G)

  h2 = (jnp.dot(emb, w0a_ref[...], preferred_element_type=_F32)
        + jnp.dot(ncc, w0b_ref[...], preferred_element_type=_F32)
        + jnp.dot(nto * validc, w0c_ref[...], preferred_element_type=_F32)
        + multi * w0d_ref[0:1]
        + nd_mean * w0d_ref[1:2]
        + sizes_sel * w0d_ref[2:3]
        + b0_ref[...])
  h2 = jnp.maximum(h2, 0.0)
  h2 = jnp.maximum(
      jnp.dot(h2, w1_ref[...], preferred_element_type=_F32) + b1_ref[...],
      0.0)
  h2 = jnp.maximum(
      jnp.dot(h2, w2_ref[...], preferred_element_type=_F32) + b2_ref[...],
      0.0)
  o_ref[...] = (jnp.dot(h2, w3_ref[...], preferred_element_type=_F32)
                + b3_ref[...])


def _final(pooled, params):
  emb = _emb_kernel(pooled, params)

  centers = emb[:_K]
  for _ in range(_KM_ITERS):
    dmat = _cdist(emb, centers)
    labels = jnp.argmin(dmat, axis=1)
    sums = jax.ops.segment_sum(emb, labels, num_segments=_K)
    counts = jax.ops.segment_sum(jnp.ones((_G,), _F32), labels,
                                 num_segments=_K)
    centers = jnp.where(counts[:, None] > 0,
                        sums / jnp.maximum(counts, 1.0)[:, None], centers)
  labels = jnp.argmin(_cdist(emb, centers), axis=1)
  sizes = jax.ops.segment_sum(jnp.ones((_G,), _F32), labels,
                              num_segments=_K)
  cdm = _cdist(centers, centers)

  kp = 8
  onehot = (labels[:, None] == jnp.arange(kp)[None, :]).astype(_F32)
  c_pad = jnp.concatenate([centers, jnp.zeros((1, _H), _F32)], axis=0)
  cdm_pad = jnp.zeros((kp, kp), _F32).at[:_K, :_K].set(cdm)
  sz_pad = jnp.concatenate([sizes, jnp.zeros((1,), _F32)]).reshape(1, kp)

  reg = params["reg"]
  w0 = reg[0]["W"]
  w0a, w0b = w0[0:_H], w0[_H:2 * _H]
  w0c = jnp.concatenate(
      [w0[2 * _H:2 * _H + _K], jnp.zeros((1, 8), _F32)], axis=0)
  w0d = w0[2 * _H + _K:2 * _H + _K + 3]
  args = [emb, onehot, c_pad, cdm_pad, sz_pad,
          w0a, w0b, w0c, w0d, reg[0]["b"].reshape(1, 8),
          reg[1]["W"], reg[1]["b"].reshape(1, 4),
          reg[2]["W"], reg[2]["b"].reshape(1, 2),
          reg[3]["W"], reg[3]["b"].reshape(1, 1)]
  full = lambda a: pl.BlockSpec(a.shape, lambda: tuple(0 for _ in a.shape))
  return pl.pallas_call(
      _feat_body,
      in_specs=[full(a) for a in args],
      out_specs=pl.BlockSpec((_G, 1), lambda: (0, 0)),
      out_shape=jax.ShapeDtypeStruct((_G, 1), _F32),
  )(*args)


def kernel(x, edge_index, batch, params):
  src = edge_index[0]
  dst = edge_index[1]
  convs = params["convs"]

  iota = jnp.arange(_E, dtype=jnp.int32)
  sd, perm = lax.sort([dst, iota], num_keys=1, is_stable=True)
  ss = src[perm]

  h = x
  width = _D
  for p in convs:
    agg = _seg_sum_sorted(h, ss, sd, _ACCN, width)
    h = _layer_mlp(h, agg, p, width)
    width = _H

  pool_ss = jnp.arange(_N, dtype=jnp.int32)
  pooled = _seg_sum_sorted(h, pool_ss, batch, _G, _H)
  return _final(pooled, params)
